# initial kernel scaffold (unmeasured)
import jax
import jax.numpy as jnp
from jax import lax
from jax.experimental import pallas as pl
from jax.experimental.pallas import tpu as pltpu

N_DEV = 32


def kernel(x, w_mat):
    m, _ = x.shape
    _, n = w_mat.shape
    chunk = m // N_DEV
    n_steps = 2 * (N_DEV - 1)

    def body(x_ref, w_ref, out_ref, comm_ref, send_sems, recv_sems, ack_sem):
        my = lax.axis_index("i")
        left = lax.rem(my + N_DEV - 1, N_DEV)
        right = lax.rem(my + 1, N_DEV)

        barrier_sem = pltpu.get_barrier_semaphore()
        for nbr in (left, right):
            pl.semaphore_signal(
                barrier_sem, inc=1,
                device_id=(nbr,), device_id_type=pl.DeviceIdType.MESH,
            )
        pl.semaphore_wait(barrier_sem, 2)

        out_ref[:, :] = jnp.dot(
            x_ref[:, :], w_ref[:, :], preferred_element_type=jnp.float32
        )

        def cs(c):
            return pl.ds(c * chunk, chunk)

        for s in range(n_steps):
            slot = s % 2
            if s < N_DEV - 1:
                send_chunk = lax.rem(my - s + N_DEV, N_DEV)
                recv_chunk = lax.rem(my - s - 1 + N_DEV, N_DEV)
            else:
                t = s - (N_DEV - 1)
                send_chunk = lax.rem(my + 1 - t + N_DEV, N_DEV)
                recv_chunk = lax.rem(my - t + N_DEV, N_DEV)

            rdma = pltpu.make_async_remote_copy(
                src_ref=out_ref.at[cs(send_chunk), :],
                dst_ref=comm_ref.at[slot],
                send_sem=send_sems.at[slot],
                recv_sem=recv_sems.at[slot],
                device_id=(right,),
                device_id_type=pl.DeviceIdType.MESH,
            )
            if s >= 2:
                pl.semaphore_wait(ack_sem, 1)
            rdma.start()
            rdma.wait_send()
            rdma.wait_recv()

            if s < N_DEV - 1:
                out_ref[cs(recv_chunk), :] += comm_ref[slot]
                if s == N_DEV - 2:
                    own = lax.rem(my + 1, N_DEV)
                    out_ref[cs(own), :] = jnp.maximum(
                        out_ref[cs(own), :], 0.0
                    )
            else:
                out_ref[cs(recv_chunk), :] = comm_ref[slot]

            if s < n_steps - 2:
                pl.semaphore_signal(
                    ack_sem, inc=1,
                    device_id=(left,), device_id_type=pl.DeviceIdType.MESH,
                )

    return pl.pallas_call(
        body,
        out_shape=jax.ShapeDtypeStruct((m, n), jnp.float32),
        in_specs=[
            pl.BlockSpec(memory_space=pltpu.VMEM),
            pl.BlockSpec(memory_space=pltpu.VMEM),
        ],
        out_specs=pl.BlockSpec(memory_space=pltpu.VMEM),
        scratch_shapes=[
            pltpu.VMEM((2, chunk, n), jnp.float32),
            pltpu.SemaphoreType.DMA((2,)),
            pltpu.SemaphoreType.DMA((2,)),
            pltpu.SemaphoreType.REGULAR,
        ],
        compiler_params=pltpu.CompilerParams(collective_id=0),
    )(x, w_mat)


# baseline (device time: 862303 ns/iter reference)
import jax
import jax.numpy as jnp
from jax import lax
from jax.experimental import pallas as pl
from jax.experimental.pallas import tpu as pltpu

N_DEV = 32


def kernel(x, w_mat):
    m, _ = x.shape
    _, n = w_mat.shape
    chunk = m // N_DEV
    n_steps = 2 * (N_DEV - 1)

    def body(x_ref, w_ref, out_ref, comm_ref, send_sems, recv_sems, ack_sem):
        my = lax.axis_index("i")
        left = lax.rem(my + N_DEV - 1, N_DEV)
        right = lax.rem(my + 1, N_DEV)

        barrier_sem = pltpu.get_barrier_semaphore()
        for nbr in (left, right):
            pl.semaphore_signal(
                barrier_sem, inc=1,
                device_id=(nbr,), device_id_type=pl.DeviceIdType.MESH,
            )
        pl.semaphore_wait(barrier_sem, 2)

        out_ref[:, :] = jnp.dot(
            x_ref[:, :], w_ref[:, :], preferred_element_type=jnp.float32
        )

        def cs(c):
            return pl.ds(c * chunk, chunk)

        for s in range(n_steps):
            slot = s % 2
            if s < N_DEV - 1:
                send_chunk = lax.rem(my - s + N_DEV, N_DEV)
                recv_chunk = lax.rem(my - s - 1 + N_DEV, N_DEV)
            else:
                t = s - (N_DEV - 1)
                send_chunk = lax.rem(my + 1 - t + N_DEV, N_DEV)
                recv_chunk = lax.rem(my - t + N_DEV, N_DEV)

            rdma = pltpu.make_async_remote_copy(
                src_ref=out_ref.at[cs(send_chunk), :],
                dst_ref=comm_ref.at[slot],
                send_sem=send_sems.at[slot],
                recv_sem=recv_sems.at[slot],
                device_id=(right,),
                device_id_type=pl.DeviceIdType.MESH,
            )
            if s >= 2:
                pl.semaphore_wait(ack_sem, 1)
            rdma.start()
            rdma.wait_send()
            rdma.wait_recv()

            if s < N_DEV - 1:
                out_ref[cs(recv_chunk), :] += comm_ref[slot]
                if s == N_DEV - 2:
                    own = lax.rem(my + 1, N_DEV)
                    out_ref[cs(own), :] = jnp.maximum(
                        out_ref[cs(own), :], 0.0
                    )
            else:
                out_ref[cs(recv_chunk), :] = comm_ref[slot]

            if s < n_steps - 2:
                pl.semaphore_signal(
                    ack_sem, inc=1,
                    device_id=(left,), device_id_type=pl.DeviceIdType.MESH,
                )

    return pl.pallas_call(
        body,
        out_shape=jax.ShapeDtypeStruct((m, n), jnp.float32),
        in_specs=[
            pl.BlockSpec(memory_space=pltpu.VMEM),
            pl.BlockSpec(memory_space=pltpu.VMEM),
        ],
        out_specs=pl.BlockSpec(memory_space=pltpu.VMEM),
        scratch_shapes=[
            pltpu.VMEM((2, chunk, n), jnp.float32),
            pltpu.SemaphoreType.DMA((2,)),
            pltpu.SemaphoreType.DMA((2,)),
            pltpu.SemaphoreType.REGULAR,
        ],
        compiler_params=pltpu.CompilerParams(
            collective_id=0,
            vmem_limit_bytes=64 * 1024 * 1024,
        ),
    )(x, w_mat)
